# speedup vs baseline: 1.3732x; 1.3732x over previous
import functools

import jax
import jax.numpy as jnp
from jax.experimental import pallas as pl
from jax.experimental.pallas import tpu as pltpu

N_NODES = 10000
D = 256
BN = 1000


def _mm_att_body(x_ref, w_ref, att2_ref, h_ref, a_ref):
    h = jnp.dot(x_ref[...], w_ref[...], preferred_element_type=jnp.float32)
    h_ref[...] = h
    a_ref[...] = jnp.dot(h, att2_ref[...], preferred_element_type=jnp.float32)


def _mm_att(x, W, att_src, att_dst):
    att2 = jnp.zeros((D, 8), jnp.float32)
    att2 = att2.at[:, 0].set(att_src).at[:, 1].set(att_dst)
    grid = N_NODES // BN
    h, a2 = pl.pallas_call(
        _mm_att_body,
        grid=(grid,),
        in_specs=[
            pl.BlockSpec((BN, D), lambda i: (i, 0)),
            pl.BlockSpec((D, D), lambda i: (0, 0)),
            pl.BlockSpec((D, 8), lambda i: (0, 0)),
        ],
        out_specs=[
            pl.BlockSpec((BN, D), lambda i: (i, 0)),
            pl.BlockSpec((BN, 8), lambda i: (i, 0)),
        ],
        out_shape=[
            jax.ShapeDtypeStruct((N_NODES, D), jnp.float32),
            jax.ShapeDtypeStruct((N_NODES, 8), jnp.float32),
        ],
    )(x, W, att2)
    return h, a2[:, 0], a2[:, 1]


def _edge_phase(h, a_src_v, a_dst_v, src, dst):
    alpha = a_src_v[src] + a_dst_v[dst]
    alpha = jax.nn.leaky_relu(alpha, negative_slope=0.2)
    gmax = jax.nn.leaky_relu(jnp.max(a_src_v) + jnp.max(a_dst_v))
    ex = jnp.exp(alpha - gmax)
    denom = jax.ops.segment_sum(ex, dst, num_segments=N_NODES)
    num = jax.ops.segment_sum(ex[:, None] * h[src], dst, num_segments=N_NODES)
    return num / (denom[:, None] + 1e-16)


def kernel(x, edge_index, W1, att_src1, att_dst1, b1, W2, att_src2, att_dst2, b2):
    loop = jnp.arange(N_NODES, dtype=edge_index.dtype)
    src = jnp.concatenate([edge_index[0], loop])
    dst = jnp.concatenate([edge_index[1], loop])

    h1, as1, ad1 = _mm_att(x, W1, att_src1, att_dst1)
    o1 = _edge_phase(h1, as1, ad1, src, dst) + b1
    o1 = jax.nn.relu(o1)
    h2, as2, ad2 = _mm_att(o1, W2, att_src2, att_dst2)
    o2 = _edge_phase(h2, as2, ad2, src, dst) + b2
    return o2


# speedup vs baseline: 8.4843x; 6.1784x over previous
import functools

import jax
import jax.numpy as jnp
from jax import lax
from jax.experimental import pallas as pl
from jax.experimental.pallas import tpu as pltpu
from jax.experimental.pallas import tpu_sc as plsc

N_NODES = 10000
D = 256
DQ = 64
BN = 1000
N_TILES = 16
E_TOT = 160000 + N_NODES
EDGE_BATCH = 128
N_BATCH = 84
CHUNK = N_BATCH * EDGE_BATCH
E_PAD = N_TILES * CHUNK
NPAD = 10240
ZROWS = NPAD // N_TILES
EPI = 128
BN2 = 1024


def _mm1_body(x_ref, w_ref, att2_ref, q0, q1, q2, q3, a2_ref):
    h = jnp.dot(x_ref[...], w_ref[...], preferred_element_type=jnp.float32)
    q0[...] = h[:, 0 * DQ:1 * DQ]
    q1[...] = h[:, 1 * DQ:2 * DQ]
    q2[...] = h[:, 2 * DQ:3 * DQ]
    q3[...] = h[:, 3 * DQ:4 * DQ]
    a2_ref[...] = jnp.dot(h, att2_ref[...], preferred_element_type=jnp.float32)


def _mm1(x, W, att2):
    return pl.pallas_call(
        _mm1_body,
        grid=(N_NODES // BN,),
        in_specs=[
            pl.BlockSpec((BN, D), lambda i: (i, 0)),
            pl.BlockSpec((D, D), lambda i: (0, 0)),
            pl.BlockSpec((D, 8), lambda i: (0, 0)),
        ],
        out_specs=[pl.BlockSpec((BN, DQ), lambda i: (i, 0))] * 4
        + [pl.BlockSpec((BN, 8), lambda i: (i, 0))],
        out_shape=[jax.ShapeDtypeStruct((N_NODES, DQ), jnp.float32)] * 4
        + [jax.ShapeDtypeStruct((N_NODES, 8), jnp.float32)],
    )(x, W, att2)


def _mm2_body(x0, x1, x2, x3, w_ref, att2_ref, q0, q1, q2, q3, a2_ref):
    h = jnp.dot(x0[...], w_ref[0 * DQ:1 * DQ, :], preferred_element_type=jnp.float32)
    h += jnp.dot(x1[...], w_ref[1 * DQ:2 * DQ, :], preferred_element_type=jnp.float32)
    h += jnp.dot(x2[...], w_ref[2 * DQ:3 * DQ, :], preferred_element_type=jnp.float32)
    h += jnp.dot(x3[...], w_ref[3 * DQ:4 * DQ, :], preferred_element_type=jnp.float32)
    q0[...] = h[:, 0 * DQ:1 * DQ]
    q1[...] = h[:, 1 * DQ:2 * DQ]
    q2[...] = h[:, 2 * DQ:3 * DQ]
    q3[...] = h[:, 3 * DQ:4 * DQ]
    a2_ref[...] = jnp.dot(h, att2_ref[...], preferred_element_type=jnp.float32)


def _mm2(xq, W, att2):
    return pl.pallas_call(
        _mm2_body,
        grid=(NPAD // BN2,),
        in_specs=[pl.BlockSpec((BN2, DQ), lambda i: (i, 0))] * 4
        + [
            pl.BlockSpec((D, D), lambda i: (0, 0)),
            pl.BlockSpec((D, 8), lambda i: (0, 0)),
        ],
        out_specs=[pl.BlockSpec((BN2, DQ), lambda i: (i, 0))] * 4
        + [pl.BlockSpec((BN2, 8), lambda i: (i, 0))],
        out_shape=[jax.ShapeDtypeStruct((NPAD, DQ), jnp.float32)] * 4
        + [jax.ShapeDtypeStruct((NPAD, 8), jnp.float32)],
    )(*xq, W, att2)


def _edge_body(relu, hq0, hq1, hq2, hq3, asrc_h, adst_h, srcb_h, dstb_h,
               gmax_h, bias_h, oq0, oq1, oq2, oq3,
               src_v, dst_v, w_v, asrc_v, adst_v, rows_v, wrow_v, gmax_v,
               bias_v, obuf, dbuf, num_sh, den_sh, sem):
    c = lax.axis_index("c")
    s = lax.axis_index("s")

    pltpu.sync_copy(srcb_h.at[s], src_v)
    pltpu.sync_copy(dstb_h.at[s], dst_v)
    pltpu.sync_copy(asrc_h, asrc_v)
    pltpu.sync_copy(adst_h, adst_v)
    pltpu.sync_copy(gmax_h, gmax_v)

    zero16 = jnp.zeros((16,), jnp.float32)
    lanes = lax.iota(jnp.int32, 16)
    zbase = s * ZROWS

    def _zrow(r, _):
        for k in range(DQ // 16):
            obuf[r, pl.ds(k * 16, 16)] = zero16
        dbuf[r, :] = zero16
        return 0

    lax.fori_loop(0, 128, _zrow, 0)

    for q in range(2):
        @pl.when(c == 0)
        def _():
            pltpu.sync_copy(bias_h.at[pl.ds(q * DQ, DQ)], bias_v)

        @pl.when(c == 1)
        def _():
            pltpu.sync_copy(bias_h.at[pl.ds((2 + q) * DQ, DQ)], bias_v)

        for k in range(ZROWS // 128):
            pltpu.sync_copy(obuf, num_sh.at[pl.ds(zbase + k * 128, 128)])
            if q == 0:
                pltpu.sync_copy(dbuf, den_sh.at[pl.ds(zbase + k * 128, 128)])
        plsc.subcore_barrier()

        gv = gmax_v[...]

        def _batch(j, _):
            @pl.when(c == 0)
            def _():
                if q == 0:
                    pltpu.async_copy(hq0.at[src_v.at[j]], rows_v, sem).wait()
                else:
                    pltpu.async_copy(hq1.at[src_v.at[j]], rows_v, sem).wait()

            @pl.when(c == 1)
            def _():
                if q == 0:
                    pltpu.async_copy(hq2.at[src_v.at[j]], rows_v, sem).wait()
                else:
                    pltpu.async_copy(hq3.at[src_v.at[j]], rows_v, sem).wait()

            if q == 0:
                eid0 = (s * CHUNK + j * EDGE_BATCH)
                for k in range(8):
                    sv = src_v[j, pl.ds(k * 16, 16)]
                    dv = dst_v[j, pl.ds(k * 16, 16)]
                    al = (plsc.load_gather(asrc_v, [sv])
                          + plsc.load_gather(adst_v, [dv]))
                    al = jnp.where(al >= 0.0, al, al * 0.2)
                    wv = jnp.exp(al - gv)
                    eid = jnp.full((16,), eid0 + k * 16, jnp.int32) + lanes
                    wv = jnp.where(eid < E_TOT, wv, 0.0)
                    w_v[j, pl.ds(k * 16, 16)] = wv

            def _srow(r, _):
                wv = plsc.load_gather(
                    w_v,
                    [jnp.full((16,), j, jnp.int32), jnp.full((16,), r, jnp.int32)])
                if q == 0:
                    wrow_v[r, :] = wv
                for k in range(DQ // 16):
                    rows_v[r, pl.ds(k * 16, 16)] = rows_v[r, pl.ds(k * 16, 16)] * wv
                return 0

            lax.fori_loop(0, EDGE_BATCH, _srow, 0)

            pltpu.sync_copy(rows_v, num_sh.at[dst_v.at[j]], add=True)
            if q == 0:
                pltpu.sync_copy(wrow_v, den_sh.at[dst_v.at[j]], add=True)
            return 0

        lax.fori_loop(0, N_BATCH, _batch, 0)
        plsc.subcore_barrier()

        for t in range(ZROWS // EPI):
            ro = zbase + t * EPI
            pltpu.sync_copy(num_sh.at[pl.ds(ro, EPI)], obuf)
            pltpu.sync_copy(den_sh.at[pl.ds(ro, EPI)], dbuf)

            def _erow(r, _):
                rv = 1.0 / (dbuf[r, :] + 1e-16)
                for k in range(DQ // 16):
                    o = obuf[r, pl.ds(k * 16, 16)] * rv + bias_v[pl.ds(k * 16, 16)]
                    if relu:
                        o = jnp.maximum(o, 0.0)
                    obuf[r, pl.ds(k * 16, 16)] = o
                return 0

            lax.fori_loop(0, EPI, _erow, 0)

            @pl.when(c == 0)
            def _():
                if q == 0:
                    pltpu.sync_copy(obuf, oq0.at[pl.ds(ro, EPI)])
                else:
                    pltpu.sync_copy(obuf, oq1.at[pl.ds(ro, EPI)])

            @pl.when(c == 1)
            def _():
                if q == 0:
                    pltpu.sync_copy(obuf, oq2.at[pl.ds(ro, EPI)])
                else:
                    pltpu.sync_copy(obuf, oq3.at[pl.ds(ro, EPI)])

        if q == 0:
            lax.fori_loop(0, 128, _zrow, 0)
            plsc.subcore_barrier()


def _edge_sc(hq, asrc, adst, srcb, dstb, gmax16, bias, relu):
    mesh = plsc.VectorSubcoreMesh(core_axis_name="c", subcore_axis_name="s")
    f32 = jnp.float32
    kern = functools.partial(
        pl.kernel,
        mesh=mesh,
        compiler_params=pltpu.CompilerParams(
            needs_layout_passes=False, use_tc_tiling_on_sc=False),
        out_type=[jax.ShapeDtypeStruct((NPAD, DQ), f32)] * 4,
        scratch_types=[
            pltpu.VMEM((N_BATCH, EDGE_BATCH), jnp.int32),
            pltpu.VMEM((N_BATCH, EDGE_BATCH), jnp.int32),
            pltpu.VMEM((N_BATCH, EDGE_BATCH), f32),
            pltpu.VMEM((N_NODES,), f32),
            pltpu.VMEM((N_NODES,), f32),
            pltpu.VMEM((EDGE_BATCH, DQ), f32),
            pltpu.VMEM((EDGE_BATCH, 16), f32),
            pltpu.VMEM((16,), f32),
            pltpu.VMEM((DQ,), f32),
            pltpu.VMEM((128, DQ), f32),
            pltpu.VMEM((128, 16), f32),
            pltpu.VMEM_SHARED((NPAD, DQ), f32),
            pltpu.VMEM_SHARED((NPAD, 16), f32),
            pltpu.SemaphoreType.DMA,
        ],
    )(functools.partial(_edge_body, relu))
    return kern(*hq, asrc, adst, srcb, dstb, gmax16, bias)


def _gmax16(asrc, adst):
    m = jnp.max(asrc) + jnp.max(adst)
    m = jnp.where(m >= 0.0, m, m * 0.2)
    return jnp.full((16,), m, jnp.float32)


def kernel(x, edge_index, W1, att_src1, att_dst1, b1, W2, att_src2, att_dst2, b2):
    loop = jnp.arange(N_NODES, dtype=edge_index.dtype)
    padi = jnp.zeros((E_PAD - E_TOT,), edge_index.dtype)
    srcb = jnp.concatenate([edge_index[0], loop, padi]).reshape(
        N_TILES, N_BATCH, EDGE_BATCH)
    dstb = jnp.concatenate([edge_index[1], loop, padi]).reshape(
        N_TILES, N_BATCH, EDGE_BATCH)

    att2_1 = jnp.zeros((D, 8), jnp.float32).at[:, 0].set(att_src1).at[:, 1].set(att_dst1)
    att2_2 = jnp.zeros((D, 8), jnp.float32).at[:, 0].set(att_src2).at[:, 1].set(att_dst2)

    *hq, a2 = _mm1(x, W1, att2_1)
    asrc, adst = a2[:, 0], a2[:, 1]
    o1 = _edge_sc(hq, asrc, adst, srcb, dstb, _gmax16(asrc, adst), b1, relu=True)

    *h2q, a2b = _mm2(o1, W2, att2_2)
    asrc2, adst2 = a2b[:N_NODES, 0], a2b[:N_NODES, 1]
    o2 = _edge_sc(h2q, asrc2, adst2, srcb, dstb, _gmax16(asrc2, adst2), b2,
                  relu=False)
    return jnp.concatenate([q[:N_NODES] for q in o2], axis=1)
